# baseline (device time: 108589 ns/iter reference)
import jax
import jax.numpy as jnp
from jax import lax
from jax.experimental import pallas as pl
from jax.experimental.pallas import tpu as pltpu

N_DEV = 16
H = N_DEV // 2
N_SEG = 2


def kernel(x, w_mat, scale_x, scale_w):
    m_per, k = x.shape
    _, n_per = w_mat.shape
    m_total = m_per * N_DEV
    m_seg = m_per // N_SEG

    fp8 = jnp.float8_e5m2

    r_set = {(h, j) for h in range(H) for j in range(N_SEG)
             if h < H - 1 or j == 0}
    l_set = {(h, j) for h in range(H) for j in range(N_SEG)
             if h < H - 1 or j == 1}

    def body(x_ref, w_ref, sx_ref, sw_ref, out_ref,
             x8, w8, comm_r, comm_l, send_r, recv_r, send_l, recv_l):
        my = lax.axis_index("i")
        right = lax.rem(my + 1, N_DEV)
        left = lax.rem(my + N_DEV - 1, N_DEV)

        barrier = pltpu.get_barrier_semaphore()
        pl.semaphore_signal(barrier, inc=1, device_id=(left,),
                            device_id_type=pl.DeviceIdType.MESH)
        pl.semaphore_signal(barrier, inc=1, device_id=(right,),
                            device_id_type=pl.DeviceIdType.MESH)

        x8[...] = x_ref[...].astype(fp8)

        pl.semaphore_wait(barrier, 2)

        def make_rdma(src, dst, ssem, rsem, tgt):
            return pltpu.make_async_remote_copy(
                src_ref=src, dst_ref=dst, send_sem=ssem, recv_sem=rsem,
                device_id=(tgt,), device_id_type=pl.DeviceIdType.MESH,
            )

        def rdma_r(h, j):
            src = x8.at[pl.ds(j * m_seg, m_seg)] if h == 0 \
                else comm_r.at[h - 1, j]
            return make_rdma(src, comm_r.at[h, j],
                             send_r.at[h, j], recv_r.at[h, j], right)

        def rdma_l(h, j):
            src = x8.at[pl.ds(j * m_seg, m_seg)] if h == 0 \
                else comm_l.at[h - 1, j]
            return make_rdma(src, comm_l.at[h, j],
                             send_l.at[h, j], recv_l.at[h, j], left)

        rdmas_r = {hj: rdma_r(*hj) for hj in r_set}
        rdmas_l = {hj: rdma_l(*hj) for hj in l_set}

        for j in range(N_SEG):
            rdmas_r[(0, j)].start()
            rdmas_l[(0, j)].start()

        w8[...] = w_ref[...].astype(fp8)
        scale = sx_ref[0] * sw_ref[0]

        def matmul_store(chunk, row_base):
            acc = lax.dot_general(
                chunk, w8[...], (((1,), (0,)), ((), ())),
                preferred_element_type=jnp.float32,
            )
            out_ref[pl.ds(row_base, m_seg), :] = jnp.maximum(
                acc * scale, 0.0)

        for j in range(N_SEG):
            matmul_store(x8[pl.ds(j * m_seg, m_seg)],
                         my * m_per + j * m_seg)

        for h in range(H):
            origin_r = lax.rem(my + N_DEV - 1 - h, N_DEV)
            origin_l = lax.rem(my + 1 + h, N_DEV)
            for j in range(N_SEG):
                if (h, j) in r_set:
                    rdmas_r[(h, j)].wait_recv()
                    if (h + 1, j) in r_set:
                        rdmas_r[(h + 1, j)].start()
                if (h, j) in l_set:
                    rdmas_l[(h, j)].wait_recv()
                    if (h + 1, j) in l_set:
                        rdmas_l[(h + 1, j)].start()
                if (h, j) in r_set:
                    matmul_store(comm_r[h, j],
                                 origin_r * m_per + j * m_seg)
                if (h, j) in l_set:
                    matmul_store(comm_l[h, j],
                                 origin_l * m_per + j * m_seg)

        for hj in sorted(r_set):
            rdmas_r[hj].wait_send()
        for hj in sorted(l_set):
            rdmas_l[hj].wait_send()

    return pl.pallas_call(
        body,
        out_shape=jax.ShapeDtypeStruct((m_total, n_per), jnp.float32),
        in_specs=[
            pl.BlockSpec(memory_space=pltpu.VMEM),
            pl.BlockSpec(memory_space=pltpu.VMEM),
            pl.BlockSpec(memory_space=pltpu.SMEM),
            pl.BlockSpec(memory_space=pltpu.SMEM),
        ],
        out_specs=pl.BlockSpec(memory_space=pltpu.VMEM),
        scratch_shapes=[
            pltpu.VMEM((m_per, k), fp8),
            pltpu.VMEM((k, n_per), fp8),
            pltpu.VMEM((H, N_SEG, m_seg, k), fp8),
            pltpu.VMEM((H, N_SEG, m_seg, k), fp8),
            pltpu.SemaphoreType.DMA((H, N_SEG)),
            pltpu.SemaphoreType.DMA((H, N_SEG)),
            pltpu.SemaphoreType.DMA((H, N_SEG)),
            pltpu.SemaphoreType.DMA((H, N_SEG)),
        ],
        compiler_params=pltpu.CompilerParams(
            collective_id=0,
            vmem_limit_bytes=100 * 1024 * 1024,
        ),
    )(x, w_mat, scale_x, scale_w)


# device time: 95944 ns/iter; 1.1318x vs baseline; 1.1318x over previous
import jax
import jax.numpy as jnp
from jax import lax
from jax.experimental import pallas as pl
from jax.experimental.pallas import tpu as pltpu

N_DEV = 16
NP = 4
NQ = 4
N_EVT = 7
OWN = 0


def ZB(s):
    return 1 + s


def ZA(s):
    return 4 + s


def kernel(x, w_mat, scale_x, scale_w):
    m_per, k = x.shape
    _, n_per = w_mat.shape
    m_total = m_per * N_DEV
    m_seg = m_per // 2

    fp8 = jnp.float8_e5m2

    def body(x_ref, w_ref, sx_ref, sw_ref, out_ref,
             x8, w8, zb, za, bl1, br1, bl2, br2,
             szu, rzb, szd, rza,
             sR1, rl1, sL1, rr1, sR2, rl2, sL2, rr2):
        my = lax.axis_index("i")
        q = lax.rem(my, NQ)
        pz = lax.div(my, NQ)
        right_n = my - q + lax.rem(q + 1, NQ)
        left_n = my - q + lax.rem(q + 3, NQ)
        up_n = lax.min(my + NQ, N_DEV - 1)
        down_n = lax.max(my - NQ, 0)

        def cond(e):
            if e == OWN:
                return None
            if e < 4:
                return pz >= (e - 1) + 1
            return pz <= 2 - (e - 4)
        def o_plane(e):
            if e == OWN:
                return pz
            if e < 4:
                return pz - 1 - (e - 1)
            return pz + 1 + (e - 4)
        def src_buf(e):
            if e == OWN:
                return x8
            if e < 4:
                return zb.at[e - 1]
            return za.at[e - 4]

        def run_if(c, fn):
            if c is None:
                fn()
            else:
                pl.when(c)(fn)

        x8[0] = x_ref[pl.ds(0, m_seg), :].astype(fp8)
        x8[1] = x_ref[pl.ds(m_seg, m_seg), :].astype(fp8)

        def make_rdma(src, dst, ssem, rsem, tgt):
            return pltpu.make_async_remote_copy(
                src_ref=src, dst_ref=dst, send_sem=ssem, recv_sem=rsem,
                device_id=(tgt,), device_id_type=pl.DeviceIdType.MESH,
            )

        dzu = [make_rdma(x8 if s == 0 else zb.at[s - 1], zb.at[s],
                         szu.at[s], rzb.at[s], up_n) for s in range(3)]
        dzd = [make_rdma(x8 if s == 0 else za.at[s - 1], za.at[s],
                         szd.at[s], rza.at[s], down_n) for s in range(3)]
        dR1 = [make_rdma(src_buf(e), bl1.at[e], sR1.at[e], rl1.at[e],
                         right_n) for e in range(N_EVT)]
        dL1 = [make_rdma(src_buf(e), br1.at[e], sL1.at[e], rr1.at[e],
                         left_n) for e in range(N_EVT)]
        dR2 = [make_rdma(bl1.at[e, 0], bl2.at[e], sR2.at[e], rl2.at[e],
                         right_n) for e in range(N_EVT)]
        dL2 = [make_rdma(br1.at[e, 1], br2.at[e], sL2.at[e], rr2.at[e],
                         left_n) for e in range(N_EVT)]

        run_if(pz <= 2, dzu[0].start)
        run_if(pz >= 1, dzd[0].start)
        dR1[OWN].start()
        dL1[OWN].start()

        w8[...] = w_ref[...].astype(fp8)
        scale = sx_ref[0] * sw_ref[0]

        def gemm_seg(chunk, row_base):
            acc = lax.dot_general(
                chunk, w8[...], (((1,), (0,)), ((), ())),
                preferred_element_type=jnp.float32,
            )
            out_ref[pl.ds(row_base, m_seg), :] = jnp.maximum(
                acc * scale, 0.0)

        def gemm_chunk(buf2, pos):
            gemm_seg(buf2[0], pos * m_per)
            gemm_seg(buf2[1], pos * m_per + m_seg)

        gemm_chunk(x8, NQ * pz + q)

        col_l = lax.rem(q + 3, NQ)
        col_r = lax.rem(q + 1, NQ)
        col_2 = lax.rem(q + 2, NQ)

        def d1_block(e):
            def _l():
                dR1[e].wait_recv()
                dR2[e].start()
                gemm_chunk(bl1.at[e], NQ * o_plane(e) + col_l)
            def _r():
                dL1[e].wait_recv()
                dL2[e].start()
                gemm_chunk(br1.at[e], NQ * o_plane(e) + col_r)
            run_if(cond(e), _l)
            run_if(cond(e), _r)

        for s in range(3):
            def _zb(s=s):
                dzu[s].wait_recv()
                if s + 1 < 3:
                    run_if(pz <= 2, dzu[s + 1].start)
                dR1[ZB(s)].start()
                dL1[ZB(s)].start()
                gemm_chunk(zb.at[s], NQ * (pz - 1 - s) + q)
            def _za(s=s):
                dzd[s].wait_recv()
                if s + 1 < 3:
                    run_if(pz >= 1, dzd[s + 1].start)
                dR1[ZA(s)].start()
                dL1[ZA(s)].start()
                gemm_chunk(za.at[s], NQ * (pz + 1 + s) + q)
            run_if(cond(ZB(s)), _zb)
            run_if(cond(ZA(s)), _za)
            if s == 0:
                d1_block(OWN)
            else:
                d1_block(ZB(s - 1))
                d1_block(ZA(s - 1))
        d1_block(ZB(2))
        d1_block(ZA(2))

        for e in range(N_EVT):
            def _d2(e=e):
                dR2[e].wait_recv()
                gemm_seg(bl2[e], (NQ * o_plane(e) + col_2) * m_per)
                dL2[e].wait_recv()
                gemm_seg(br2[e],
                         (NQ * o_plane(e) + col_2) * m_per + m_seg)
            run_if(cond(e), _d2)

        run_if(pz <= 2, dzu[0].wait_send)
        run_if(pz >= 1, dzd[0].wait_send)
        for s in range(1, 3):
            run_if(jnp.logical_and(pz >= s, pz <= 2), dzu[s].wait_send)
            run_if(jnp.logical_and(pz >= 1, pz <= 3 - s),
                   dzd[s].wait_send)
        for e in range(N_EVT):
            for d in (dR1, dL1, dR2, dL2):
                run_if(cond(e), d[e].wait_send)

    return pl.pallas_call(
        body,
        out_shape=jax.ShapeDtypeStruct((m_total, n_per), jnp.float32),
        in_specs=[
            pl.BlockSpec(memory_space=pltpu.VMEM),
            pl.BlockSpec(memory_space=pltpu.VMEM),
            pl.BlockSpec(memory_space=pltpu.SMEM),
            pl.BlockSpec(memory_space=pltpu.SMEM),
        ],
        out_specs=pl.BlockSpec(memory_space=pltpu.VMEM),
        scratch_shapes=[
            pltpu.VMEM((2, m_seg, k), fp8),
            pltpu.VMEM((k, n_per), fp8),
            pltpu.VMEM((3, 2, m_seg, k), fp8),
            pltpu.VMEM((3, 2, m_seg, k), fp8),
            pltpu.VMEM((N_EVT, 2, m_seg, k), fp8),
            pltpu.VMEM((N_EVT, 2, m_seg, k), fp8),
            pltpu.VMEM((N_EVT, m_seg, k), fp8),
            pltpu.VMEM((N_EVT, m_seg, k), fp8),
            pltpu.SemaphoreType.DMA((3,)),
            pltpu.SemaphoreType.DMA((3,)),
            pltpu.SemaphoreType.DMA((3,)),
            pltpu.SemaphoreType.DMA((3,)),
            pltpu.SemaphoreType.DMA((N_EVT,)),
            pltpu.SemaphoreType.DMA((N_EVT,)),
            pltpu.SemaphoreType.DMA((N_EVT,)),
            pltpu.SemaphoreType.DMA((N_EVT,)),
            pltpu.SemaphoreType.DMA((N_EVT,)),
            pltpu.SemaphoreType.DMA((N_EVT,)),
            pltpu.SemaphoreType.DMA((N_EVT,)),
            pltpu.SemaphoreType.DMA((N_EVT,)),
        ],
        compiler_params=pltpu.CompilerParams(
            vmem_limit_bytes=100 * 1024 * 1024,
        ),
    )(x, w_mat, scale_x, scale_w)


# device time: 90331 ns/iter; 1.2021x vs baseline; 1.0621x over previous
import jax
import jax.numpy as jnp
from jax import lax
from jax.experimental import pallas as pl
from jax.experimental.pallas import tpu as pltpu

N_DEV = 16
NP = 4
NQ = 4
N_EVT = 7
OWN = 0


def ZB(s):
    return 1 + s


def ZA(s):
    return 4 + s


def kernel(x, w_mat, scale_x, scale_w):
    m_per, k = x.shape
    _, n_per = w_mat.shape
    m_total = m_per * N_DEV
    m_seg = m_per // 2

    fp8 = jnp.float8_e5m2

    def body(x_ref, w_ref, sx_ref, sw_ref, out_ref,
             x8, w8, zb, za, bl1, br1, bl2, br2,
             szu, rzb, szd, rza,
             sR1, rl1, sL1, rr1, sR2, rl2, sL2, rr2):
        my = lax.axis_index("i")
        q = lax.rem(my, NQ)
        pz = lax.div(my, NQ)
        right_n = my - q + lax.rem(q + 1, NQ)
        left_n = my - q + lax.rem(q + 3, NQ)
        up_n = lax.min(my + NQ, N_DEV - 1)
        down_n = lax.max(my - NQ, 0)

        def cond(e):
            if e == OWN:
                return None
            if e < 4:
                return pz >= (e - 1) + 1
            return pz <= 2 - (e - 4)
        def o_plane(e):
            if e == OWN:
                return pz
            if e < 4:
                return pz - 1 - (e - 1)
            return pz + 1 + (e - 4)
        def src_buf(e):
            if e == OWN:
                return x8
            if e < 4:
                return zb.at[e - 1]
            return za.at[e - 4]

        def run_if(c, fn):
            if c is None:
                fn()
            else:
                pl.when(c)(fn)

        barrier = pltpu.get_barrier_semaphore()
        for nbr in (left_n, right_n):
            pl.semaphore_signal(barrier, inc=1, device_id=(nbr,),
                                device_id_type=pl.DeviceIdType.MESH)
        run_if(pz <= 2, lambda: pl.semaphore_signal(
            barrier, inc=1, device_id=(up_n,),
            device_id_type=pl.DeviceIdType.MESH))
        run_if(pz >= 1, lambda: pl.semaphore_signal(
            barrier, inc=1, device_id=(down_n,),
            device_id_type=pl.DeviceIdType.MESH))

        x8[0] = x_ref[pl.ds(0, m_seg), :].astype(fp8)
        x8[1] = x_ref[pl.ds(m_seg, m_seg), :].astype(fp8)

        pl.semaphore_wait(barrier, 2)
        run_if(pz <= 2, lambda: pl.semaphore_wait(barrier, 1))
        run_if(pz >= 1, lambda: pl.semaphore_wait(barrier, 1))

        def make_rdma(src, dst, ssem, rsem, tgt):
            return pltpu.make_async_remote_copy(
                src_ref=src, dst_ref=dst, send_sem=ssem, recv_sem=rsem,
                device_id=(tgt,), device_id_type=pl.DeviceIdType.MESH,
            )

        dzu = [make_rdma(x8 if s == 0 else zb.at[s - 1], zb.at[s],
                         szu.at[s], rzb.at[s], up_n) for s in range(3)]
        dzd = [make_rdma(x8 if s == 0 else za.at[s - 1], za.at[s],
                         szd.at[s], rza.at[s], down_n) for s in range(3)]
        dR1 = [make_rdma(src_buf(e), bl1.at[e], sR1.at[e], rl1.at[e],
                         right_n) for e in range(N_EVT)]
        dL1 = [make_rdma(src_buf(e), br1.at[e], sL1.at[e], rr1.at[e],
                         left_n) for e in range(N_EVT)]
        dR2 = [make_rdma(bl1.at[e, 0], bl2.at[e], sR2.at[e], rl2.at[e],
                         right_n) for e in range(N_EVT)]
        dL2 = [make_rdma(br1.at[e, 1], br2.at[e], sL2.at[e], rr2.at[e],
                         left_n) for e in range(N_EVT)]

        run_if(pz <= 2, dzu[0].start)
        run_if(pz >= 1, dzd[0].start)
        dR1[OWN].start()
        dL1[OWN].start()

        w8[...] = w_ref[...].astype(fp8)
        scale = sx_ref[0] * sw_ref[0]

        def gemm_seg(chunk, row_base):
            acc = lax.dot_general(
                chunk, w8[...], (((1,), (0,)), ((), ())),
                preferred_element_type=jnp.float32,
            )
            out_ref[pl.ds(row_base, m_seg), :] = jnp.maximum(
                acc * scale, 0.0)

        def gemm_chunk(buf2, pos):
            gemm_seg(buf2[0], pos * m_per)
            gemm_seg(buf2[1], pos * m_per + m_seg)

        gemm_chunk(x8, NQ * pz + q)

        col_l = lax.rem(q + 3, NQ)
        col_r = lax.rem(q + 1, NQ)
        col_2 = lax.rem(q + 2, NQ)

        def d1_block(e):
            def _l():
                dR1[e].wait_recv()
                dR2[e].start()
                gemm_chunk(bl1.at[e], NQ * o_plane(e) + col_l)
            def _r():
                dL1[e].wait_recv()
                dL2[e].start()
                gemm_chunk(br1.at[e], NQ * o_plane(e) + col_r)
            run_if(cond(e), _l)
            run_if(cond(e), _r)

        for s in range(3):
            def _zb(s=s):
                dzu[s].wait_recv()
                if s + 1 < 3:
                    run_if(pz <= 2, dzu[s + 1].start)
                dR1[ZB(s)].start()
                dL1[ZB(s)].start()
                gemm_chunk(zb.at[s], NQ * (pz - 1 - s) + q)
            def _za(s=s):
                dzd[s].wait_recv()
                if s + 1 < 3:
                    run_if(pz >= 1, dzd[s + 1].start)
                dR1[ZA(s)].start()
                dL1[ZA(s)].start()
                gemm_chunk(za.at[s], NQ * (pz + 1 + s) + q)
            run_if(cond(ZB(s)), _zb)
            run_if(cond(ZA(s)), _za)
            if s == 0:
                d1_block(OWN)
            else:
                d1_block(ZB(s - 1))
                d1_block(ZA(s - 1))
        d1_block(ZB(2))
        d1_block(ZA(2))

        for e in range(N_EVT):
            def _d2(e=e):
                dR2[e].wait_recv()
                gemm_seg(bl2[e], (NQ * o_plane(e) + col_2) * m_per)
                dL2[e].wait_recv()
                gemm_seg(br2[e],
                         (NQ * o_plane(e) + col_2) * m_per + m_seg)
            run_if(cond(e), _d2)

        run_if(pz <= 2, dzu[0].wait_send)
        run_if(pz >= 1, dzd[0].wait_send)
        for s in range(1, 3):
            run_if(jnp.logical_and(pz >= s, pz <= 2), dzu[s].wait_send)
            run_if(jnp.logical_and(pz >= 1, pz <= 3 - s),
                   dzd[s].wait_send)
        for e in range(N_EVT):
            for d in (dR1, dL1, dR2, dL2):
                run_if(cond(e), d[e].wait_send)

    return pl.pallas_call(
        body,
        out_shape=jax.ShapeDtypeStruct((m_total, n_per), jnp.float32),
        in_specs=[
            pl.BlockSpec(memory_space=pltpu.VMEM),
            pl.BlockSpec(memory_space=pltpu.VMEM),
            pl.BlockSpec(memory_space=pltpu.SMEM),
            pl.BlockSpec(memory_space=pltpu.SMEM),
        ],
        out_specs=pl.BlockSpec(memory_space=pltpu.VMEM),
        scratch_shapes=[
            pltpu.VMEM((2, m_seg, k), fp8),
            pltpu.VMEM((k, n_per), fp8),
            pltpu.VMEM((3, 2, m_seg, k), fp8),
            pltpu.VMEM((3, 2, m_seg, k), fp8),
            pltpu.VMEM((N_EVT, 2, m_seg, k), fp8),
            pltpu.VMEM((N_EVT, 2, m_seg, k), fp8),
            pltpu.VMEM((N_EVT, m_seg, k), fp8),
            pltpu.VMEM((N_EVT, m_seg, k), fp8),
            pltpu.SemaphoreType.DMA((3,)),
            pltpu.SemaphoreType.DMA((3,)),
            pltpu.SemaphoreType.DMA((3,)),
            pltpu.SemaphoreType.DMA((3,)),
            pltpu.SemaphoreType.DMA((N_EVT,)),
            pltpu.SemaphoreType.DMA((N_EVT,)),
            pltpu.SemaphoreType.DMA((N_EVT,)),
            pltpu.SemaphoreType.DMA((N_EVT,)),
            pltpu.SemaphoreType.DMA((N_EVT,)),
            pltpu.SemaphoreType.DMA((N_EVT,)),
            pltpu.SemaphoreType.DMA((N_EVT,)),
            pltpu.SemaphoreType.DMA((N_EVT,)),
        ],
        compiler_params=pltpu.CompilerParams(
            collective_id=0,
            vmem_limit_bytes=100 * 1024 * 1024,
        ),
    )(x, w_mat, scale_x, scale_w)
